# baseline (speedup 1.0000x reference)
import jax
import jax.numpy as jnp
from jax.experimental import pallas as pl
from jax.experimental.pallas import tpu as pltpu

B, S, D, E = 4, 8192, 2048, 64
CHUNK = 256
NPAR = 2


def _sum_kernel(x_ref, out_ref, acc_ref):
    j = pl.program_id(1)
    nj = pl.num_programs(1)

    @pl.when(j == 0)
    def _init():
        acc_ref[...] = jnp.zeros_like(acc_ref)

    acc_ref[...] += jnp.sum(x_ref[...], axis=1)

    @pl.when(j == nj - 1)
    def _fin():
        out_ref[...] = acc_ref[...][None]


def _combine_kernel(p_ref, w_ref, b_ref, out_ref):
    s = jnp.sum(p_ref[...], axis=0)
    logits = jax.lax.dot_general(
        s, w_ref[...],
        dimension_numbers=(((1,), (1,)), ((), ())),
        preferred_element_type=jnp.float32,
    ) + b_ref[...]
    out_ref[...] = jnp.argmax(logits, axis=1).astype(jnp.int32)[None, :]


def kernel(x, W, b):
    nj = S // (NPAR * CHUNK)
    partials = pl.pallas_call(
        _sum_kernel,
        grid=(NPAR, nj),
        in_specs=[
            pl.BlockSpec((B, CHUNK, D), lambda h, j: (0, h * nj + j, 0)),
        ],
        out_specs=pl.BlockSpec((1, B, D), lambda h, j: (h, 0, 0)),
        out_shape=jax.ShapeDtypeStruct((NPAR, B, D), jnp.float32),
        scratch_shapes=[pltpu.VMEM((B, D), jnp.float32)],
        compiler_params=pltpu.CompilerParams(
            dimension_semantics=("parallel", "arbitrary"),
        ),
    )(x)
    out = pl.pallas_call(
        _combine_kernel,
        in_specs=[
            pl.BlockSpec((NPAR, B, D), lambda: (0, 0, 0)),
            pl.BlockSpec((E, D), lambda: (0, 0)),
            pl.BlockSpec((1, E), lambda: (0, 0)),
        ],
        out_specs=pl.BlockSpec((1, B), lambda: (0, 0)),
        out_shape=jax.ShapeDtypeStruct((1, B), jnp.int32),
    )(partials, W, b.reshape(1, E))
    return out.reshape(B)
